# baseline (speedup 1.0000x reference)
import functools

import jax
import jax.numpy as jnp
from jax import lax
from jax.experimental import pallas as pl
from jax.experimental.pallas import tpu as pltpu
from jax.experimental.pallas import tpu_sc as plsc

SIGMA = 1.0
B = 8
N = 2048
CI = 256
CJ = 256
NI = N // CI
NJ = N // CJ
L = 16
SCLAMP = 44.0

def _sc_partition_body(scores_hbm, rel_hbm, pos_hbm, neg_hbm, cnt_hbm,
                       s_v, r_v, pos_v, neg_v, cnt_v):
    wid = lax.axis_index("s") * 2 + lax.axis_index("c")

    @pl.when(wid < B)
    def _():
        b = wid
        pltpu.sync_copy(scores_hbm.at[b], s_v)
        pltpu.sync_copy(rel_hbm.at[b], r_v)

        def initloop(i, carry):
            pos_v[pl.ds(i * L, L)] = jnp.zeros((L,), jnp.float32)
            neg_v[pl.ds(i * L, L)] = jnp.zeros((L,), jnp.float32)
            return carry

        lax.fori_loop(0, N // L, initloop, 0)

        lane = lax.iota(jnp.int32, L)

        def chunk(i, carry):
            off_p, off_n = carry
            s = s_v[pl.ds(i * L, L)]
            r = r_v[pl.ds(i * L, L)]
            m = r == 1
            mi = jnp.where(m, 1, 0).astype(jnp.int32)
            incl = plsc.cumsum(mi)
            excl = incl - mi
            npos = jnp.sum(mi)
            pos_idx = lax.broadcast(off_p, (L,)) + excl
            neg_idx = lax.broadcast(off_n, (L,)) + (lane - excl)
            sc = jnp.minimum(jnp.maximum(s, -SCLAMP), SCLAMP) * SIGMA
            plsc.store_scatter(pos_v, [pos_idx], jnp.exp(-sc), mask=m)
            plsc.store_scatter(neg_v, [neg_idx], jnp.exp(sc),
                              mask=jnp.logical_not(m))
            return off_p + npos, off_n + (L - npos)

        off_p, _ = lax.fori_loop(0, N // L, chunk,
                                 (jnp.int32(0), jnp.int32(0)))
        cnt_v[...] = lax.broadcast(off_p, (L,))
        pltpu.sync_copy(pos_v, pos_hbm.at[b])
        pltpu.sync_copy(neg_v, neg_hbm.at[b])
        pltpu.sync_copy(cnt_v, cnt_hbm.at[b])


_sc_partition_fn = None


def _sc_partition(scores, rel):
    global _sc_partition_fn
    if _sc_partition_fn is None:
        mesh = plsc.VectorSubcoreMesh(
            core_axis_name="c", subcore_axis_name="s",
            num_cores=2, num_subcores=16)
        _sc_partition_fn = pl.kernel(
            _sc_partition_body,
            compiler_params=pltpu.CompilerParams(needs_layout_passes=False),
            out_type=[
                jax.ShapeDtypeStruct((B, N), jnp.float32),
                jax.ShapeDtypeStruct((B, N), jnp.float32),
                jax.ShapeDtypeStruct((B, L), jnp.int32),
            ],
            mesh=mesh,
            scratch_types=[
                pltpu.VMEM((N,), jnp.float32),
                pltpu.VMEM((N,), jnp.int32),
                pltpu.VMEM((N,), jnp.float32),
                pltpu.VMEM((N,), jnp.float32),
                pltpu.VMEM((L,), jnp.int32),
            ],
        )
    return _sc_partition_fn(scores, rel)


def _pair_kernel(cnt_ref, pos_t_ref, neg_rs_ref, loss_ref, pairs_ref):
    acc = jnp.zeros((8, 128), jnp.float32)
    npairs = jnp.float32(0.0)
    for b in range(B):
        p = cnt_ref[b, 0]
        q = N - p
        npairs += (p * q).astype(jnp.float32)
        ni = (p + CI - 1) // CI
        nj = (q + CJ - 1) // CJ

        def iloop(ci, acc_i, b=b, nj=nj):
            rows = pos_t_ref[pl.ds(ci * CI, CI), b:b + 1]

            def jloop(cj, acc2, rows=rows, b=b):
                cols = neg_rs_ref[pl.ds(b * NJ + cj, 1)]
                l2 = jnp.log2(1.0 + cols.reshape(1, CJ) * rows)
                part = jnp.sum(l2.reshape(CI // 8, 8, CJ // 128, 128),
                               axis=(0, 2))
                return acc2 + part

            return lax.fori_loop(0, nj, jloop, acc_i)

        acc = lax.fori_loop(0, ni, iloop, acc)
    loss_ref[0, 0] = jnp.sum(acc) * jnp.float32(0.6931471805599453)
    pairs_ref[0, 0] = npairs


def _pairwise_call(cnt, pos_t, neg_rs):
    return pl.pallas_call(
        _pair_kernel,
        in_specs=[
            pl.BlockSpec(memory_space=pltpu.SMEM),
            pl.BlockSpec(memory_space=pltpu.VMEM),
            pl.BlockSpec(memory_space=pltpu.VMEM),
        ],
        out_specs=[
            pl.BlockSpec(memory_space=pltpu.SMEM),
            pl.BlockSpec(memory_space=pltpu.SMEM),
        ],
        out_shape=[
            jax.ShapeDtypeStruct((1, 1), jnp.float32),
            jax.ShapeDtypeStruct((1, 1), jnp.float32),
        ],
    )(cnt, pos_t, neg_rs)


def kernel(scores, relevances):
    rel = relevances.astype(jnp.int32)
    pos, neg, cnt = _sc_partition(scores, rel)
    pos_t = pos.T
    neg_rs = neg.reshape(B * NJ, 1, CJ)
    loss, pairs = _pairwise_call(cnt, pos_t, neg_rs)
    total = loss[0, 0]
    npr = pairs[0, 0]
    return jnp.where(npr > 0, total / npr, total)


# speedup vs baseline: 1.5673x; 1.5673x over previous
import functools

import jax
import jax.numpy as jnp
from jax import lax
from jax.experimental import pallas as pl
from jax.experimental.pallas import tpu as pltpu
from jax.experimental.pallas import tpu_sc as plsc

SIGMA = 1.0
B = 8
N = 2048
CI = 256
CJ = 512
NI = N // CI
NJ = N // CJ
L = 16
SCLAMP = 44.0

def _sc_partition_body(scores_hbm, rel_hbm, pos_hbm, neg_hbm, cnt_hbm,
                       s_v, r_v, pos_v, neg_v, cnt_v):
    wid = lax.axis_index("s") * 2 + lax.axis_index("c")

    @pl.when(wid < B)
    def _():
        b = wid
        pltpu.sync_copy(scores_hbm.at[b], s_v)
        pltpu.sync_copy(rel_hbm.at[b], r_v)

        def initloop(i, carry):
            pos_v[pl.ds(i * L, L)] = jnp.zeros((L,), jnp.float32)
            neg_v[pl.ds(i * L, L)] = jnp.zeros((L,), jnp.float32)
            return carry

        lax.fori_loop(0, N // L, initloop, 0)

        lane = lax.iota(jnp.int32, L)

        def chunk(i, carry):
            off_p, off_n = carry
            s = s_v[pl.ds(i * L, L)]
            r = r_v[pl.ds(i * L, L)]
            m = r == 1
            mi = jnp.where(m, 1, 0).astype(jnp.int32)
            incl = plsc.cumsum(mi)
            excl = incl - mi
            npos = jnp.sum(mi)
            pos_idx = lax.broadcast(off_p, (L,)) + excl
            neg_idx = lax.broadcast(off_n, (L,)) + (lane - excl)
            sc = jnp.minimum(jnp.maximum(s, -SCLAMP), SCLAMP) * SIGMA
            plsc.store_scatter(pos_v, [pos_idx], jnp.exp(-sc), mask=m)
            plsc.store_scatter(neg_v, [neg_idx], jnp.exp(sc),
                              mask=jnp.logical_not(m))
            return off_p + npos, off_n + (L - npos)

        off_p, _ = lax.fori_loop(0, N // L, chunk,
                                 (jnp.int32(0), jnp.int32(0)))
        cnt_v[...] = lax.broadcast(off_p, (L,))
        pltpu.sync_copy(pos_v, pos_hbm.at[b])
        pltpu.sync_copy(neg_v, neg_hbm.at[b])
        pltpu.sync_copy(cnt_v, cnt_hbm.at[b])


_sc_partition_fn = None


def _sc_partition(scores, rel):
    global _sc_partition_fn
    if _sc_partition_fn is None:
        mesh = plsc.VectorSubcoreMesh(
            core_axis_name="c", subcore_axis_name="s",
            num_cores=2, num_subcores=16)
        _sc_partition_fn = pl.kernel(
            _sc_partition_body,
            compiler_params=pltpu.CompilerParams(needs_layout_passes=False),
            out_type=[
                jax.ShapeDtypeStruct((B, N), jnp.float32),
                jax.ShapeDtypeStruct((B, N), jnp.float32),
                jax.ShapeDtypeStruct((B, L), jnp.int32),
            ],
            mesh=mesh,
            scratch_types=[
                pltpu.VMEM((N,), jnp.float32),
                pltpu.VMEM((N,), jnp.int32),
                pltpu.VMEM((N,), jnp.float32),
                pltpu.VMEM((N,), jnp.float32),
                pltpu.VMEM((L,), jnp.int32),
            ],
        )
    return _sc_partition_fn(scores, rel)


def _pair_kernel(cnt_ref, pos_t_ref, neg_rs_ref, loss_ref, pairs_ref,
                 acc_ref):
    acc_ref[...] = jnp.zeros((CI, CJ), jnp.float32)
    npairs = jnp.float32(0.0)
    for b in range(B):
        p = cnt_ref[b, 0]
        q = N - p
        npairs += (p * q).astype(jnp.float32)
        ni = (p + CI - 1) // CI
        nj = (q + CJ - 1) // CJ

        def iloop(ci, acc_i, b=b, nj=nj):
            rows = pos_t_ref[pl.ds(ci * CI, CI), b:b + 1]

            def jloop(cj, acc2, rows=rows, b=b):
                cols = neg_rs_ref[pl.ds(b * NJ + cj, 1)]
                l2 = jnp.log2(1.0 + cols.reshape(1, CJ) * rows)
                acc_ref[...] += l2
                return acc2

            return lax.fori_loop(0, nj, jloop, acc_i)

        lax.fori_loop(0, ni, iloop, 0)
    loss_ref[0, 0] = jnp.sum(acc_ref[...]) * jnp.float32(0.6931471805599453)
    pairs_ref[0, 0] = npairs


def _pairwise_call(cnt, pos_t, neg_rs):
    return pl.pallas_call(
        _pair_kernel,
        in_specs=[
            pl.BlockSpec(memory_space=pltpu.SMEM),
            pl.BlockSpec(memory_space=pltpu.VMEM),
            pl.BlockSpec(memory_space=pltpu.VMEM),
        ],
        out_specs=[
            pl.BlockSpec(memory_space=pltpu.SMEM),
            pl.BlockSpec(memory_space=pltpu.SMEM),
        ],
        out_shape=[
            jax.ShapeDtypeStruct((1, 1), jnp.float32),
            jax.ShapeDtypeStruct((1, 1), jnp.float32),
        ],
        scratch_shapes=[pltpu.VMEM((CI, CJ), jnp.float32)],
    )(cnt, pos_t, neg_rs)


def kernel(scores, relevances):
    rel = relevances.astype(jnp.int32)
    pos, neg, cnt = _sc_partition(scores, rel)
    pos_t = pos.T
    neg_rs = neg.reshape(B * NJ, 1, CJ)
    loss, pairs = _pairwise_call(cnt, pos_t, neg_rs)
    total = loss[0, 0]
    npr = pairs[0, 0]
    return jnp.where(npr > 0, total / npr, total)


# speedup vs baseline: 1.5706x; 1.0021x over previous
import jax
import jax.numpy as jnp
from jax import lax
from jax.experimental import pallas as pl
from jax.experimental.pallas import tpu as pltpu
from jax.experimental.pallas import tpu_sc as plsc

SIGMA = 1.0
B = 8
N = 2048
CI = 256
CJ = 512
NI = N // CI
NJ = N // CJ
L = 16
SCLAMP = 44.0

def _sc_partition_body(scores_hbm, rel_hbm, pos_hbm, neg_hbm, cnt_hbm,
                       s_v, r_v, pos_v, neg_v, cnt_v):
    wid = lax.axis_index("s") * 2 + lax.axis_index("c")

    @pl.when(wid < B)
    def _():
        b = wid
        pltpu.sync_copy(scores_hbm.at[b], s_v)
        pltpu.sync_copy(rel_hbm.at[b], r_v)

        def initloop(i, carry):
            pos_v[pl.ds(i * L, L)] = jnp.zeros((L,), jnp.float32)
            neg_v[pl.ds(i * L, L)] = jnp.zeros((L,), jnp.float32)
            return carry

        lax.fori_loop(0, N // L, initloop, 0)

        lane = lax.iota(jnp.int32, L)

        def chunk(i, carry):
            off_p, off_n = carry
            s = s_v[pl.ds(i * L, L)]
            r = r_v[pl.ds(i * L, L)]
            m = r == 1
            mi = jnp.where(m, 1, 0).astype(jnp.int32)
            incl = plsc.cumsum(mi)
            excl = incl - mi
            npos = jnp.sum(mi)
            pos_idx = lax.broadcast(off_p, (L,)) + excl
            neg_idx = lax.broadcast(off_n, (L,)) + (lane - excl)
            sc = jnp.minimum(jnp.maximum(s, -SCLAMP), SCLAMP) * SIGMA
            plsc.store_scatter(pos_v, [pos_idx], jnp.exp(-sc), mask=m)
            plsc.store_scatter(neg_v, [neg_idx], jnp.exp(sc),
                              mask=jnp.logical_not(m))
            return off_p + npos, off_n + (L - npos)

        off_p, _ = lax.fori_loop(0, N // L, chunk,
                                 (jnp.int32(0), jnp.int32(0)))
        cnt_v[...] = lax.broadcast(off_p, (L,))
        pltpu.sync_copy(pos_v, pos_hbm.at[b])
        pltpu.sync_copy(neg_v, neg_hbm.at[b])
        pltpu.sync_copy(cnt_v, cnt_hbm.at[b])


_sc_partition_fn = None


def _sc_partition(scores, rel):
    global _sc_partition_fn
    if _sc_partition_fn is None:
        mesh = plsc.VectorSubcoreMesh(
            core_axis_name="c", subcore_axis_name="s",
            num_cores=2, num_subcores=16)
        _sc_partition_fn = pl.kernel(
            _sc_partition_body,
            compiler_params=pltpu.CompilerParams(needs_layout_passes=False),
            out_type=[
                jax.ShapeDtypeStruct((B, N), jnp.float32),
                jax.ShapeDtypeStruct((B, N), jnp.float32),
                jax.ShapeDtypeStruct((B, L), jnp.int32),
            ],
            mesh=mesh,
            scratch_types=[
                pltpu.VMEM((N,), jnp.float32),
                pltpu.VMEM((N,), jnp.int32),
                pltpu.VMEM((N,), jnp.float32),
                pltpu.VMEM((N,), jnp.float32),
                pltpu.VMEM((L,), jnp.int32),
            ],
        )
    return _sc_partition_fn(scores, rel)


def _pair_kernel(cnt_ref, pos_t_ref, neg_rs_ref, loss_ref, pairs_ref,
                 acc_ref):
    acc_ref[...] = jnp.zeros((CI, CJ), jnp.float32)
    npairs = jnp.float32(0.0)
    for b in range(B):
        p = cnt_ref[b, 0]
        q = N - p
        npairs += (p * q).astype(jnp.float32)
        ni = (p + CI - 1) // CI
        nj = (q + CJ - 1) // CJ

        def iloop(ci, acc_i, b=b, nj=nj):
            rows = pos_t_ref[pl.ds(ci * CI, CI), b:b + 1]

            def jloop(cj, acc2, rows=rows, b=b):
                cols = neg_rs_ref[pl.ds(b * NJ + cj, 1)]
                l2 = jnp.log2(1.0 + cols.reshape(1, CJ) * rows)
                acc_ref[...] += l2
                return acc2

            return lax.fori_loop(0, nj, jloop, acc_i)

        lax.fori_loop(0, ni, iloop, 0)
    loss_ref[0, 0] = jnp.sum(acc_ref[...]) * jnp.float32(0.6931471805599453)
    pairs_ref[0, 0] = npairs


def _pairwise_call(cnt, pos_t, neg_rs):
    return pl.pallas_call(
        _pair_kernel,
        in_specs=[
            pl.BlockSpec(memory_space=pltpu.SMEM),
            pl.BlockSpec(memory_space=pltpu.VMEM),
            pl.BlockSpec(memory_space=pltpu.VMEM),
        ],
        out_specs=[
            pl.BlockSpec(memory_space=pltpu.SMEM),
            pl.BlockSpec(memory_space=pltpu.SMEM),
        ],
        out_shape=[
            jax.ShapeDtypeStruct((1, 1), jnp.float32),
            jax.ShapeDtypeStruct((1, 1), jnp.float32),
        ],
        scratch_shapes=[pltpu.VMEM((CI, CJ), jnp.float32)],
    )(cnt, pos_t, neg_rs)


def kernel(scores, relevances):
    rel = relevances.astype(jnp.int32)
    pos, neg, cnt = _sc_partition(scores, rel)
    pos_t = pos.T
    neg_rs = neg.reshape(B * NJ, 1, CJ)
    loss, pairs = _pairwise_call(cnt, pos_t, neg_rs)
    total = loss[0, 0]
    npr = pairs[0, 0]
    return jnp.where(npr > 0, total / npr, total)
